# baseline (speedup 1.0000x reference)
import jax
import jax.numpy as jnp
import numpy as np
from jax import lax
from jax.experimental import pallas as pl
from jax.experimental.pallas import tpu as pltpu
from jax.experimental.pallas import tpu_sc as plsc

N = 10000
C = 128
K = 32
HIDDEN = 128
TW = 160
BN = 4
NBLK = N // BN
NW = 32
TL = 80
NBLK_PAD = NW * TL
LANES = 16
TCH = TW // 32

_PERM = np.empty(C, np.int32)
for _c in range(C // 32):
  for _q in range(32):
    _PERM[32 * _c + _q] = 32 * _c + (2 * _q if _q < 16 else 2 * (_q - 16) + 1)


def _sc_gather_body(knn_hbm, tab_hbm, aggp_hbm,
                    idx_all, rows0, rows1, agg0, agg1,
                    semg0, semg1, semo0, semo1):
  cid = lax.axis_index("c")
  sid = lax.axis_index("s")
  wid = sid * 2 + cid
  base = wid * TL

  rows_b = (rows0, rows1)
  agg_b = (agg0, agg1)
  semg = (semg0, semg1)
  semo = (semo0, semo1)

  pltpu.sync_copy(knn_hbm.at[pl.ds(base, TL)], idx_all)

  def gather(t, p):
    return pltpu.make_async_copy(tab_hbm.at[idx_all.at[t]], rows_b[p], semg[p])

  def out_copy(t, p):
    return pltpu.make_async_copy(
        agg_b[p], aggp_hbm.at[pl.ds((base + t) * BN, BN)], semo[p])

  gather(0, 0).start()

  def outer(g, _):
    for phase in range(2):
      t = g * 2 + phase
      tn = t + 1

      @pl.when((tn < TL) & (base + tn < NBLK))
      def _():
        gather(tn, 1 - phase).start()

      @pl.when(base + t < NBLK)
      def _():
        @pl.when(t >= 2)
        def _():
          out_copy(t - 2, phase).wait()

        gather(t, phase).wait()

        rows_v = rows_b[phase]
        for n in range(BN):
          def red(j, carry):
            r = n * K + j
            new = []
            for c in range(TCH):
              e, o = plsc.unpack(rows_v[r, pl.ds(32 * c, 32)],
                                 format=plsc.PackFormat.INTERLEAVED)
              new.append(carry[2 * c] + e)
              new.append(carry[2 * c + 1] + o)
            return tuple(new)

          zero = jnp.zeros((LANES,), jnp.float32)
          init = tuple(zero for _ in range(2 * TCH))
          out = lax.fori_loop(0, K, red, init)
          for c in range(TCH):
            agg_b[phase][n, pl.ds(32 * c, LANES)] = out[2 * c]
            agg_b[phase][n, pl.ds(32 * c + LANES, LANES)] = out[2 * c + 1]

        out_copy(t, phase).start()

    return _

  lax.fori_loop(0, TL // 2, outer, None)

  for p in range(2):
    out_copy(0, p).wait()


def _sc_gather(knn2d, tab):
  mesh = plsc.VectorSubcoreMesh(core_axis_name="c", subcore_axis_name="s")
  return pl.kernel(
      _sc_gather_body,
      out_type=jax.ShapeDtypeStruct((N, TW), jnp.float32),
      mesh=mesh,
      compiler_params=pltpu.CompilerParams(use_tc_tiling_on_sc=False,
                                           needs_layout_passes=False),
      scratch_types=[
          pltpu.VMEM((TL, BN * K), jnp.int32),
          pltpu.VMEM((BN * K, TW), jnp.bfloat16),
          pltpu.VMEM((BN * K, TW), jnp.bfloat16),
          pltpu.VMEM((BN, TW), jnp.float32),
          pltpu.VMEM((BN, TW), jnp.float32),
          pltpu.SemaphoreType.DMA,
          pltpu.SemaphoreType.DMA,
          pltpu.SemaphoreType.DMA,
          pltpu.SemaphoreType.DMA,
      ],
  )(knn2d, tab)


def _tc_dense_body(feat_ref, aggp_ref, cp_ref,
                   w1_ref, w2_ref, wm_ref, ws_ref, b_ref, out_ref):
  inv_k = 1.0 / K
  aggp = aggp_ref[...]
  m1 = aggp[:, C:C + LANES] * inv_k
  m2 = aggp[:, C + LANES:C + 2 * LANES] * inv_k
  rm = m1 - cp_ref[...]
  rs = jnp.sqrt(jnp.maximum(m2 - m1 * m1, 0.0))
  acc = jnp.dot(feat_ref[...], w1_ref[...], preferred_element_type=jnp.float32)
  acc += jnp.dot(aggp[:, 0:C], w2_ref[...], preferred_element_type=jnp.float32)
  acc += jnp.dot(rm, wm_ref[...], preferred_element_type=jnp.float32)
  acc += jnp.dot(rs, ws_ref[...], preferred_element_type=jnp.float32)
  acc += b_ref[...]
  out_ref[...] = acc * jax.nn.sigmoid(acc)


def _tc_dense(feat, aggp, cpad, w1, w2p, wm, ws, b2):
  R = 1000
  grid = (N // R,)
  return pl.pallas_call(
      _tc_dense_body,
      grid=grid,
      in_specs=[
          pl.BlockSpec((R, C), lambda i: (i, 0)),
          pl.BlockSpec((R, TW), lambda i: (i, 0)),
          pl.BlockSpec((R, LANES), lambda i: (i, 0)),
          pl.BlockSpec((C, HIDDEN), lambda i: (0, 0)),
          pl.BlockSpec((C, HIDDEN), lambda i: (0, 0)),
          pl.BlockSpec((LANES, HIDDEN), lambda i: (0, 0)),
          pl.BlockSpec((LANES, HIDDEN), lambda i: (0, 0)),
          pl.BlockSpec((1, HIDDEN), lambda i: (0, 0)),
      ],
      out_specs=pl.BlockSpec((R, HIDDEN), lambda i: (i, 0)),
      out_shape=jax.ShapeDtypeStruct((N, HIDDEN), jnp.float32),
  )(feat, aggp, cpad, w1, w2p, wm, ws, b2)


def kernel(feat, coords, knn_idx, W, b):
  knn2d = jnp.zeros((NBLK_PAD, BN * K), jnp.int32).at[:NBLK].set(
      knn_idx.astype(jnp.int32).reshape(NBLK, BN * K))
  cc = jnp.stack([coords, coords * coords], axis=-1).reshape(N, 6)
  tab = (jnp.zeros((N, TW), jnp.bfloat16)
         .at[:, :C].set(feat.astype(jnp.bfloat16))
         .at[:, C:C + 6].set(cc.astype(jnp.bfloat16)))
  aggp = _sc_gather(knn2d, tab)

  cpad = jnp.zeros((N, LANES), jnp.float32).at[:, :3].set(coords)
  w1 = W[0:C]
  w2p = (W[C:2 * C] * (1.0 / K))[_PERM]
  wm = jnp.zeros((LANES, HIDDEN), jnp.float32).at[0:3].set(W[2 * C:2 * C + 3])
  ws = jnp.zeros((LANES, HIDDEN), jnp.float32).at[0:3].set(
      W[2 * C + 3:2 * C + 6])
  b2 = b.reshape(1, HIDDEN)
  return _tc_dense(feat, aggp, cpad, w1, w2p, wm, ws, b2)


# speedup vs baseline: 1.3206x; 1.3206x over previous
import jax
import jax.numpy as jnp
import numpy as np
from jax import lax
from jax.experimental import pallas as pl
from jax.experimental.pallas import tpu as pltpu
from jax.experimental.pallas import tpu_sc as plsc

N = 10000
C = 128
K = 32
HIDDEN = 128
ST = 16
BN = 4
NBLK = N // BN
NW = 32
TL = 80
LANES = 16
CCH = C // 32

_PERM = np.empty(C, np.int32)
for _c in range(CCH):
  for _q in range(32):
    _PERM[32 * _c + _q] = 32 * _c + (2 * _q if _q < 16 else 2 * (_q - 16) + 1)


def _sc_gather_body(knn_hbm, feat_hbm, stat_hbm, aggp_hbm, ssum_hbm,
                    idx_all, rows0, rows1, srows0, srows1,
                    agg0, agg1, st0, st1, semg0, semg1, semo0, semo1):
  cid = lax.axis_index("c")
  sid = lax.axis_index("s")
  wid = sid * 2 + cid
  base = jnp.minimum(wid * TL, NBLK - TL)

  rows_b = (rows0, rows1)
  srows_b = (srows0, srows1)
  agg_b = (agg0, agg1)
  st_b = (st0, st1)
  semg = (semg0, semg1)
  semo = (semo0, semo1)

  pltpu.sync_copy(knn_hbm.at[pl.ds(base, TL)], idx_all)

  def gathers(t, p):
    return (
        pltpu.make_async_copy(feat_hbm.at[idx_all.at[t]], rows_b[p], semg[p]),
        pltpu.make_async_copy(stat_hbm.at[idx_all.at[t]], srows_b[p], semg[p]),
    )

  def out_copies(t, p):
    return (
        pltpu.make_async_copy(
            agg_b[p], aggp_hbm.at[pl.ds((base + t) * BN, BN)], semo[p]),
        pltpu.make_async_copy(
            st_b[p], ssum_hbm.at[pl.ds((base + t) * BN, BN)], semo[p]),
    )

  for cp in gathers(0, 0):
    cp.start()

  def outer(g, _):
    for phase in range(2):
      t = g * 2 + phase
      tn = t + 1

      @pl.when(tn < TL)
      def _():
        for cp in gathers(tn, 1 - phase):
          cp.start()

      @pl.when(t >= 2)
      def _():
        for cp in out_copies(t - 2, phase):
          cp.wait()

      for cp in gathers(t, phase):
        cp.wait()

      rows_v = rows_b[phase]
      srows_v = srows_b[phase]
      for n in range(BN):
        def red(j, carry):
          r = n * K + j
          new = []
          for c in range(CCH):
            e, o = plsc.unpack(rows_v[r, pl.ds(32 * c, 32)],
                               format=plsc.PackFormat.INTERLEAVED)
            new.append(carry[2 * c] + e)
            new.append(carry[2 * c + 1] + o)
          new.append(carry[2 * CCH] + srows_v[r, :])
          return tuple(new)

        zero = jnp.zeros((LANES,), jnp.float32)
        init = tuple(zero for _ in range(2 * CCH + 1))
        out = lax.fori_loop(0, K, red, init)
        for c in range(CCH):
          agg_b[phase][n, pl.ds(32 * c, LANES)] = out[2 * c]
          agg_b[phase][n, pl.ds(32 * c + LANES, LANES)] = out[2 * c + 1]
        st_b[phase][n, :] = out[2 * CCH]

      for cp in out_copies(t, phase):
        cp.start()

    return _

  lax.fori_loop(0, TL // 2, outer, None)

  for p in range(2):
    for cp in out_copies(0, p):
      cp.wait()


def _sc_gather(knn2d, featb, stat16):
  mesh = plsc.VectorSubcoreMesh(core_axis_name="c", subcore_axis_name="s")
  return pl.kernel(
      _sc_gather_body,
      out_type=(
          jax.ShapeDtypeStruct((N, C), jnp.float32),
          jax.ShapeDtypeStruct((N, ST), jnp.float32),
      ),
      mesh=mesh,
      compiler_params=pltpu.CompilerParams(use_tc_tiling_on_sc=False,
                                           needs_layout_passes=False),
      scratch_types=[
          pltpu.VMEM((TL, BN * K), jnp.int32),
          pltpu.VMEM((BN * K, C), jnp.bfloat16),
          pltpu.VMEM((BN * K, C), jnp.bfloat16),
          pltpu.VMEM((BN * K, ST), jnp.float32),
          pltpu.VMEM((BN * K, ST), jnp.float32),
          pltpu.VMEM((BN, C), jnp.float32),
          pltpu.VMEM((BN, C), jnp.float32),
          pltpu.VMEM((BN, ST), jnp.float32),
          pltpu.VMEM((BN, ST), jnp.float32),
          pltpu.SemaphoreType.DMA,
          pltpu.SemaphoreType.DMA,
          pltpu.SemaphoreType.DMA,
          pltpu.SemaphoreType.DMA,
      ],
  )(knn2d, featb, stat16)


def _tc_dense_body(feat_ref, aggp_ref, ssum_ref, coords_ref,
                   w1_ref, w2_ref, w3_ref, b_ref, out_ref):
  inv_k = 1.0 / K
  s = ssum_ref[...] * inv_k
  m1 = s[:, 0:3]
  m2 = s[:, 3:6]
  rm = m1 - coords_ref[...]
  rs = jnp.sqrt(jnp.maximum(m2 - m1 * m1, 0.0))
  rel = jnp.concatenate([rm, rs], axis=-1)
  acc = jnp.dot(feat_ref[...], w1_ref[...], preferred_element_type=jnp.float32)
  acc += jnp.dot(aggp_ref[...], w2_ref[...], preferred_element_type=jnp.float32)
  acc += jnp.dot(rel, w3_ref[...], preferred_element_type=jnp.float32)
  acc += b_ref[...]
  out_ref[...] = acc * jax.nn.sigmoid(acc)


def _tc_dense(feat, aggp, ssum, coords, w1, w2p, w3, b2):
  R = 1000
  grid = (N // R,)
  return pl.pallas_call(
      _tc_dense_body,
      grid=grid,
      in_specs=[
          pl.BlockSpec((R, C), lambda i: (i, 0)),
          pl.BlockSpec((R, C), lambda i: (i, 0)),
          pl.BlockSpec((R, ST), lambda i: (i, 0)),
          pl.BlockSpec((R, 3), lambda i: (i, 0)),
          pl.BlockSpec((C, HIDDEN), lambda i: (0, 0)),
          pl.BlockSpec((C, HIDDEN), lambda i: (0, 0)),
          pl.BlockSpec((6, HIDDEN), lambda i: (0, 0)),
          pl.BlockSpec((1, HIDDEN), lambda i: (0, 0)),
      ],
      out_specs=pl.BlockSpec((R, HIDDEN), lambda i: (i, 0)),
      out_shape=jax.ShapeDtypeStruct((N, HIDDEN), jnp.float32),
  )(feat, aggp, ssum, coords, w1, w2p, w3, b2)


def kernel(feat, coords, knn_idx, W, b):
  knn2d = knn_idx.astype(jnp.int32).reshape(NBLK, BN * K)
  featb = feat.astype(jnp.bfloat16)
  stat16 = jnp.concatenate(
      [coords, coords * coords, jnp.zeros((N, ST - 6), jnp.float32)], axis=1)
  aggp, ssum = _sc_gather(knn2d, featb, stat16)

  w1 = W[0:C]
  w2p = (W[C:2 * C] * (1.0 / K))[_PERM]
  w3 = W[2 * C:2 * C + 6]
  b2 = b.reshape(1, HIDDEN)
  return _tc_dense(feat, aggp, ssum, coords, w1, w2p, w3, b2)
